# baseline (speedup 1.0000x reference)
import functools

import numpy as np
import jax
import jax.numpy as jnp
from jax import lax
from jax.experimental import pallas as pl
from jax.experimental.pallas import tpu as pltpu
from jax.experimental.pallas import tpu_sc as plsc

_MAXLEN = 200
_EMBED = 64
_BATCH = 1024
_ROWS = _BATCH * _MAXLEN
_NC, _NS, _LANES = 2, 16, 16
_NW = _NC * _NS
_BPW = _ROWS // _NW
_CHUNK = 128
_NCHUNK = _BPW // _CHUNK
_VPR = _EMBED // _LANES


def _pos_table_np():
    pos = np.arange(_MAXLEN)[:, None].astype(np.float64)
    j = np.arange(_EMBED)[None, :]
    angle = pos / np.power(10000.0, 2.0 * (j // 2) / _EMBED)
    return np.where(j % 2 == 0, np.sin(angle), np.cos(angle)).astype(np.float32)


def _sc_embed(x3, table, pos):
    mesh = plsc.VectorSubcoreMesh(core_axis_name="c", subcore_axis_name="s")

    @functools.partial(
        pl.kernel,
        mesh=mesh,
        compiler_params=pltpu.CompilerParams(use_tc_tiling_on_sc=False),
        out_type=jax.ShapeDtypeStruct((_ROWS, _EMBED), jnp.float32),
        scratch_types=[
            pltpu.VMEM((_NCHUNK, _CHUNK), jnp.int32),
            pltpu.VMEM((2 * _MAXLEN, _EMBED), jnp.float32),
            pltpu.VMEM((_CHUNK, _EMBED), jnp.float32),
            pltpu.SemaphoreType.DMA,
        ],
    )
    def k(x_hbm, tab_hbm, pos_hbm, out_hbm, idx_v, pos_v, buf, gsem):
        wid = lax.axis_index("s") * _NC + lax.axis_index("c")
        pltpu.sync_copy(x_hbm.at[wid], idx_v)
        pltpu.sync_copy(pos_hbm, pos_v)
        base = wid * _BPW

        def chunk_body(c, carry):
            pltpu.async_copy(tab_hbm.at[idx_v.at[c]], buf, gsem).wait()
            pstart = lax.rem(c * _CHUNK, _MAXLEN)

            def row_body(r, rcarry):
                for q in range(_VPR):
                    sl = pl.ds(q * _LANES, _LANES)
                    pv = pos_v[pstart + r, sl]
                    plsc.addupdate(buf.at[r, sl], pv)
                return rcarry

            lax.fori_loop(0, _CHUNK, row_body, 0)
            pltpu.sync_copy(buf, out_hbm.at[pl.ds(base + c * _CHUNK, _CHUNK)])
            return carry

        lax.fori_loop(0, _NCHUNK, chunk_body, 0)

    return k(x3, table, pos)


def kernel(x, token_table):
    pos = jnp.asarray(np.tile(_pos_table_np(), (2, 1)))
    xf = x.reshape(_NW, _NCHUNK, _CHUNK).astype(jnp.int32)
    out = _sc_embed(xf, token_table, pos)
    return out.reshape(_BATCH, _MAXLEN, _EMBED)


# speedup vs baseline: 1.0863x; 1.0863x over previous
import functools

import numpy as np
import jax
import jax.numpy as jnp
from jax import lax
from jax.experimental import pallas as pl
from jax.experimental.pallas import tpu as pltpu
from jax.experimental.pallas import tpu_sc as plsc

_MAXLEN = 200
_EMBED = 64
_BATCH = 1024
_ROWS = _BATCH * _MAXLEN
_NC, _NS, _LANES = 2, 16, 16
_NW = _NC * _NS
_BPW = _ROWS // _NW
_CHUNK = 128
_NCHUNK = _BPW // _CHUNK
_VPR = _EMBED // _LANES
_NBUF = 5
_PREF = 3


def _pos_table_np():
    pos = np.arange(_MAXLEN)[:, None].astype(np.float64)
    j = np.arange(_EMBED)[None, :]
    angle = pos / np.power(10000.0, 2.0 * (j // 2) / _EMBED)
    return np.where(j % 2 == 0, np.sin(angle), np.cos(angle)).astype(np.float32)


def _sc_embed(x3, table, pos):
    mesh = plsc.VectorSubcoreMesh(core_axis_name="c", subcore_axis_name="s")

    @functools.partial(
        pl.kernel,
        mesh=mesh,
        compiler_params=pltpu.CompilerParams(use_tc_tiling_on_sc=False),
        out_type=jax.ShapeDtypeStruct((_ROWS, _EMBED), jnp.float32),
        scratch_types=[
            pltpu.VMEM((_NCHUNK, _CHUNK), jnp.int32),
            pltpu.VMEM((2 * _MAXLEN, _EMBED), jnp.float32),
            pltpu.VMEM((_NBUF, _CHUNK, _EMBED), jnp.float32),
        ]
        + [pltpu.SemaphoreType.DMA] * (2 * _NBUF),
    )
    def k(x_hbm, tab_hbm, pos_hbm, out_hbm, idx_v, pos_v, bufs, *sems):
        gsems = sems[:_NBUF]
        osems = sems[_NBUF:]
        wid = lax.axis_index("s") * _NC + lax.axis_index("c")
        pltpu.sync_copy(x_hbm.at[wid], idx_v)
        pltpu.sync_copy(pos_hbm, pos_v)
        base = wid * _BPW

        for s in range(_PREF):
            pltpu.async_copy(tab_hbm.at[idx_v.at[s]], bufs.at[s], gsems[s])

        def group_body(g, carry):
            for s in range(_NBUF):
                c = g * _NBUF + s
                pltpu.make_async_copy(
                    tab_hbm.at[idx_v.at[0]], bufs.at[s], gsems[s]
                ).wait()

                pstart = lax.rem(c * _CHUNK, _MAXLEN)

                def row_body(it, rcarry):
                    r0 = it * 8
                    for dr in range(8):
                        for q in range(_VPR):
                            sl = pl.ds(q * _LANES, _LANES)
                            pv = pos_v[pstart + r0 + dr, sl]
                            plsc.addupdate(bufs.at[s].at[r0 + dr, sl], pv)
                    return rcarry

                lax.fori_loop(0, _CHUNK // 8, row_body, 0)

                pltpu.async_copy(
                    bufs.at[s],
                    out_hbm.at[pl.ds(base + c * _CHUNK, _CHUNK)],
                    osems[s],
                )

                p = c + _PREF
                sp = (s + _PREF) % _NBUF

                @pl.when(jnp.logical_and(c >= _NBUF - _PREF, c < _NCHUNK - _PREF))
                def _wait_slot():
                    pltpu.make_async_copy(
                        bufs.at[sp],
                        out_hbm.at[pl.ds(base, _CHUNK)],
                        osems[sp],
                    ).wait()

                @pl.when(c < _NCHUNK - _PREF)
                def _prefetch():
                    pltpu.async_copy(
                        tab_hbm.at[idx_v.at[p]], bufs.at[sp], gsems[sp]
                    )

            return carry

        lax.fori_loop(0, _NCHUNK // _NBUF, group_body, 0)

        for s in range(_NBUF):
            pltpu.make_async_copy(
                bufs.at[s], out_hbm.at[pl.ds(base, _CHUNK)], osems[s]
            ).wait()

    return k(x3, table, pos)


def kernel(x, token_table):
    pos = jnp.asarray(np.tile(_pos_table_np(), (2, 1)))
    xf = x.reshape(_NW, _NCHUNK, _CHUNK).astype(jnp.int32)
    out = _sc_embed(xf, token_table, pos)
    return out.reshape(_BATCH, _MAXLEN, _EMBED)


# speedup vs baseline: 1.1772x; 1.0837x over previous
import functools

import numpy as np
import jax
import jax.numpy as jnp
from jax import lax
from jax.experimental import pallas as pl
from jax.experimental.pallas import tpu as pltpu
from jax.experimental.pallas import tpu_sc as plsc

_MAXLEN = 200
_EMBED = 64
_BATCH = 1024
_ROWS = _BATCH * _MAXLEN
_NC, _NS, _LANES = 2, 16, 16
_NW = _NC * _NS
_BPW = _ROWS // _NW
_CHUNK = 128
_NCHUNK = _BPW // _CHUNK
_VPR = _EMBED // _LANES
_NBUF = 5
_PREF = 3


def _pos_table_np():
    pos = np.arange(_MAXLEN)[:, None].astype(np.float64)
    j = np.arange(_EMBED)[None, :]
    angle = pos / np.power(10000.0, 2.0 * (j // 2) / _EMBED)
    return np.where(j % 2 == 0, np.sin(angle), np.cos(angle)).astype(np.float32)


def _sc_embed(x3, table, pos):
    mesh = plsc.VectorSubcoreMesh(core_axis_name="c", subcore_axis_name="s")

    @functools.partial(
        pl.kernel,
        mesh=mesh,
        compiler_params=pltpu.CompilerParams(use_tc_tiling_on_sc=False),
        out_type=jax.ShapeDtypeStruct((_ROWS, _EMBED), jnp.float32),
        scratch_types=[
            pltpu.VMEM((_NCHUNK, _CHUNK), jnp.int32),
            pltpu.VMEM((2 * _MAXLEN, _EMBED), jnp.float32),
            pltpu.VMEM((_NBUF, _CHUNK, _EMBED), jnp.float32),
        ]
        + [pltpu.SemaphoreType.DMA] * (2 * _NBUF),
    )
    def k(x_hbm, tab_hbm, pos_hbm, out_hbm, idx_v, pos_v, bufs, *sems):
        gsems = sems[:_NBUF]
        osems = sems[_NBUF:]
        wid = lax.axis_index("s") * _NC + lax.axis_index("c")
        pltpu.sync_copy(x_hbm.at[wid], idx_v)
        pltpu.sync_copy(pos_hbm, pos_v)
        base = wid * _BPW

        for s in range(_PREF):
            pltpu.async_copy(tab_hbm.at[idx_v.at[s]], bufs.at[s], gsems[s])

        def group_body(g, carry):
            for s in range(_NBUF):
                c = g * _NBUF + s
                pltpu.make_async_copy(
                    tab_hbm.at[idx_v.at[0]], bufs.at[s], gsems[s]
                ).wait()

                pstart = lax.rem(c * _CHUNK, _MAXLEN)

                @plsc.parallel_loop(0, _CHUNK, 1, unroll=8)
                def _row_body(r):
                    for q in range(_VPR):
                        sl = pl.ds(q * _LANES, _LANES)
                        pv = pos_v[pstart + r, sl]
                        plsc.addupdate(bufs.at[s].at[r, sl], pv)

                pltpu.async_copy(
                    bufs.at[s],
                    out_hbm.at[pl.ds(base + c * _CHUNK, _CHUNK)],
                    osems[s],
                )

                p = c + _PREF
                sp = (s + _PREF) % _NBUF

                @pl.when(jnp.logical_and(c >= _NBUF - _PREF, c < _NCHUNK - _PREF))
                def _wait_slot():
                    pltpu.make_async_copy(
                        bufs.at[sp],
                        out_hbm.at[pl.ds(base, _CHUNK)],
                        osems[sp],
                    ).wait()

                @pl.when(c < _NCHUNK - _PREF)
                def _prefetch():
                    pltpu.async_copy(
                        tab_hbm.at[idx_v.at[p]], bufs.at[sp], gsems[sp]
                    )

            return carry

        lax.fori_loop(0, _NCHUNK // _NBUF, group_body, 0)

        for s in range(_NBUF):
            pltpu.make_async_copy(
                bufs.at[s], out_hbm.at[pl.ds(base, _CHUNK)], osems[s]
            ).wait()

    return k(x3, table, pos)


def kernel(x, token_table):
    pos = jnp.asarray(np.tile(_pos_table_np(), (2, 1)))
    xf = x.reshape(_NW, _NCHUNK, _CHUNK).astype(jnp.int32)
    out = _sc_embed(xf, token_table, pos)
    return out.reshape(_BATCH, _MAXLEN, _EMBED)
